# Initial kernel scaffold; baseline (speedup 1.0000x reference)
import jax
import jax.numpy as jnp
from jax.experimental import pallas as pl


def kernel(graph_embeddings, graph_edges, sequence, WA, bA, WB, bB, WD, bD, WE, bE, W_in, conv_w, conv_b, W_xproj, W_dt, b_dt, A_log, D_param, W_out, W_ff1, b_ff1, W_ff2, b_ff2):
    raise NotImplementedError("write your pallas kernel here")


# speedup vs baseline: 2.7013x; 2.7013x over previous
import functools

import jax
import jax.numpy as jnp
from jax import lax
from jax.experimental import pallas as pl
from jax.experimental.pallas import tpu as pltpu

N_NODES = 10000
N_EDGES = 160000
DIM = 256
D_STATE = 16
D_CONV = 4
DT_RANK = 16
B_SEQ = 4
L_SEQ = 2048

_TL = 256


def _m1_body(u_ref, w_in_ref, conv_wT_ref, conv_b_ref, w_xproj_ref, w_dt_ref,
             b_dt_ref, xs_ref, z_ref, dt_ref, dtx_ref, bc_ref, tail_ref):
    l = pl.program_id(1)
    u = u_ref[0]
    xz = jnp.dot(u, w_in_ref[...], preferred_element_type=jnp.float32)
    x, z = xz[:, :DIM], xz[:, DIM:]
    z_ref[0] = z

    @pl.when(l == 0)
    def _():
        tail_ref[...] = jnp.zeros_like(tail_ref)

    xpad = jnp.concatenate([tail_ref[...], x], axis=0)
    tail_ref[...] = x[_TL - (D_CONV - 1):, :]
    xc = jnp.zeros_like(x)
    for k in range(D_CONV):
        xc = xc + xpad[k:k + _TL, :] * conv_wT_ref[k, :][None, :]
    xc = xc + conv_b_ref[0, :][None, :]
    xs = xc * jax.nn.sigmoid(xc)
    xs_ref[0] = xs

    x_dbl = jnp.dot(xs, w_xproj_ref[...], preferred_element_type=jnp.float32)
    bc_ref[0] = x_dbl[:, DT_RANK:]
    dt_pre = jnp.dot(x_dbl[:, :DT_RANK], w_dt_ref[...],
                     preferred_element_type=jnp.float32) + b_dt_ref[0, :][None, :]
    dt = jnp.maximum(dt_pre, 0.0) + jnp.log1p(jnp.exp(-jnp.abs(dt_pre)))
    dt_ref[0] = dt
    dtx_ref[0] = dt * xs


def _run_m1(u, W_in, conv_wT, conv_b, W_xproj, W_dt, b_dt):
    B, L, D = u.shape
    grid = (B, L // _TL)
    bl = lambda i, j: (i, j, 0)
    out_shapes = [
        jax.ShapeDtypeStruct((B, L, D), jnp.float32),
        jax.ShapeDtypeStruct((B, L, D), jnp.float32),
        jax.ShapeDtypeStruct((B, L, D), jnp.float32),
        jax.ShapeDtypeStruct((B, L, D), jnp.float32),
        jax.ShapeDtypeStruct((B, L, 2 * D_STATE), jnp.float32),
    ]
    out_specs = [pl.BlockSpec((1, _TL, D), bl)] * 4 + [
        pl.BlockSpec((1, _TL, 2 * D_STATE), bl)]
    full = lambda s: pl.BlockSpec(s, lambda i, j: (0,) * len(s))
    return pl.pallas_call(
        _m1_body,
        grid=grid,
        in_specs=[
            pl.BlockSpec((1, _TL, D), bl),
            full((D, 2 * D)),
            full((D_CONV, D)),
            full((1, D)),
            full((D, DT_RANK + 2 * D_STATE)),
            full((DT_RANK, D)),
            full((1, D)),
        ],
        out_specs=out_specs,
        out_shape=out_shapes,
        scratch_shapes=[pltpu.VMEM((D_CONV - 1, D), jnp.float32)],
    )(u, W_in, conv_wT, conv_b, W_xproj, W_dt, b_dt)


_T = 128


def _m2_body(dt_ref, dtx_ref, bc_ref, xs_ref, z_ref, at_ref, dparam_ref,
             y_ref, h_ref, da_ref, dbx_ref, hh_ref):
    j = pl.program_id(1)

    @pl.when(j == 0)
    def _():
        h_ref[...] = jnp.zeros_like(h_ref)

    dt = dt_ref[0]
    dtx = dtx_ref[0]
    bm = bc_ref[0][:, :D_STATE]
    cm = bc_ref[0][:, D_STATE:]
    at = at_ref[...]

    da_ref[...] = jnp.exp(dt[:, None, :] * at[None, :, :])
    dbx_ref[...] = dtx[:, None, :] * bm[:, :, None]

    def step(t, h):
        h = da_ref[t] * h + dbx_ref[t]
        hh_ref[t] = h
        return h

    h_ref[...] = lax.fori_loop(0, _T, step, h_ref[...], unroll=8)

    ysum = jnp.sum(hh_ref[...] * cm[:, :, None], axis=1)
    y = ysum + xs_ref[0] * dparam_ref[0, :][None, :]
    z = z_ref[0]
    y_ref[0] = y * (z * jax.nn.sigmoid(z))


def _run_m2(dt, dtx, bc, xs, z, AT, D_param):
    B, L, D = dt.shape
    grid = (B, L // _T)
    bl = lambda i, j: (i, j, 0)
    full = lambda s: pl.BlockSpec(s, lambda i, j: (0,) * len(s))
    return pl.pallas_call(
        _m2_body,
        grid=grid,
        in_specs=[
            pl.BlockSpec((1, _T, D), bl),
            pl.BlockSpec((1, _T, D), bl),
            pl.BlockSpec((1, _T, 2 * D_STATE), bl),
            pl.BlockSpec((1, _T, D), bl),
            pl.BlockSpec((1, _T, D), bl),
            full((D_STATE, D)),
            full((1, D)),
        ],
        out_specs=pl.BlockSpec((1, _T, D), bl),
        out_shape=jax.ShapeDtypeStruct((B, L, D), jnp.float32),
        scratch_shapes=[
            pltpu.VMEM((D_STATE, D), jnp.float32),
            pltpu.VMEM((_T, D_STATE, D), jnp.float32),
            pltpu.VMEM((_T, D_STATE, D), jnp.float32),
            pltpu.VMEM((_T, D_STATE, D), jnp.float32),
        ],
    )(dt, dtx, bc, xs, z, AT, D_param)


_RT = 512


def _m3_body(y_ref, hin_ref, h1_ref, w_out_ref, w_ff1_ref, b_ff1_ref,
             w_ff2_ref, b_ff2_ref, nv_ref):
    hin = hin_ref[...]
    h_attn = hin + jnp.dot(y_ref[...], w_out_ref[...],
                           preferred_element_type=jnp.float32)
    h = h1_ref[...] + h_attn
    t1 = jnp.maximum(
        jnp.dot(h, w_ff1_ref[...], preferred_element_type=jnp.float32)
        + b_ff1_ref[0, :][None, :], 0.0)
    hout = h + jnp.dot(t1, w_ff2_ref[...],
                       preferred_element_type=jnp.float32) + b_ff2_ref[0, :][None, :]
    nv_ref[...] = 0.5 * (hin + hout)


def _run_m3(y, h_in1, h1, W_out, W_ff1, b_ff1, W_ff2, b_ff2):
    R, D = y.shape
    grid = (R // _RT,)
    bl = lambda i: (i, 0)
    full = lambda s: pl.BlockSpec(s, lambda i: (0,) * len(s))
    return pl.pallas_call(
        _m3_body,
        grid=grid,
        in_specs=[
            pl.BlockSpec((_RT, D), bl),
            pl.BlockSpec((_RT, D), bl),
            pl.BlockSpec((_RT, D), bl),
            full((D, D)),
            full((D, 2 * D)),
            full((1, 2 * D)),
            full((2 * D, D)),
            full((1, D)),
        ],
        out_specs=pl.BlockSpec((_RT, D), bl),
        out_shape=jax.ShapeDtypeStruct((R, D), jnp.float32),
    )(y, h_in1, h1, W_out, W_ff1, b_ff1, W_ff2, b_ff2)


_NT = 512
_NPAD = 10240


def _k1_body(x_ref, w_ref, b_ref, ax_ref, bx_ref, dx_ref, ex_ref):
    out = jnp.dot(x_ref[...], w_ref[...], preferred_element_type=jnp.float32) \
        + b_ref[0, :][None, :]
    ax_ref[...] = out[:, :DIM]
    bx_ref[...] = out[:, DIM:2 * DIM]
    dx_ref[...] = out[:, 2 * DIM:3 * DIM]
    ex_ref[...] = out[:, 3 * DIM:]


def _run_k1(x, Wcat, bcat):
    Np, D = x.shape
    grid = (Np // _NT,)
    bl = lambda i: (i, 0)
    full = lambda s: pl.BlockSpec(s, lambda i: (0,) * len(s))
    shp = jax.ShapeDtypeStruct((Np, D), jnp.float32)
    return pl.pallas_call(
        _k1_body,
        grid=grid,
        in_specs=[pl.BlockSpec((_NT, D), bl), full((D, 4 * D)), full((1, 4 * D))],
        out_specs=[pl.BlockSpec((_NT, D), bl)] * 4,
        out_shape=[shp, shp, shp, shp],
    )(x, Wcat, bcat)


def kernel(graph_embeddings, graph_edges, sequence, WA, bA, WB, bB, WD, bD,
           WE, bE, W_in, conv_w, conv_b, W_xproj, W_dt, b_dt, A_log, D_param,
           W_out, W_ff1, b_ff1, W_ff2, b_ff2):
    ge = graph_embeddings
    src = graph_edges[0]
    dst = graph_edges[1]
    seq_flat = sequence.reshape(-1)

    Wcat = jnp.concatenate([WA, WB, WD, WE], axis=1)
    bcat = jnp.concatenate([bA, bB, bD, bE])[None, :]
    ge_pad = jnp.pad(ge, ((0, _NPAD - N_NODES), (0, 0)))
    Ax, Bx, Dx, Ex = _run_k1(ge_pad, Wcat, bcat)
    Ax, Bx, Dx, Ex = (a[:N_NODES] for a in (Ax, Bx, Dx, Ex))

    e = Dx[dst] + Ex[src]
    sigma = jax.nn.sigmoid(e)
    num = jax.ops.segment_sum(sigma * Bx[src], dst, num_segments=N_NODES)
    den = jax.ops.segment_sum(sigma, dst, num_segments=N_NODES)
    h_gcn = jnp.maximum(Ax + num / (den + 1e-6), 0.0)
    h_local = ge + h_gcn

    h_in1 = ge[seq_flat].reshape(B_SEQ, L_SEQ, DIM)
    h1 = h_local[seq_flat]

    conv_wT = conv_w.T
    xs, z, dt, dtx, bc = _run_m1(h_in1, W_in, conv_wT, conv_b[None, :],
                                 W_xproj, W_dt, b_dt[None, :])
    AT = (-jnp.exp(A_log)).T
    y = _run_m2(dt, dtx, bc, xs, z, AT, D_param[None, :])

    new_vals = _run_m3(y.reshape(-1, DIM), h_in1.reshape(-1, DIM), h1,
                       W_out, W_ff1, b_ff1[None, :], W_ff2, b_ff2[None, :])

    emb = ge.at[seq_flat].set(new_vals)
    return emb
